# baseline (speedup 1.0000x reference)
import functools

import jax
import jax.numpy as jnp
from jax import lax
from jax.experimental import pallas as pl
from jax.experimental.pallas import tpu as pltpu
from jax.experimental.pallas import tpu_sc as plsc

_EPS = 0.1
_NC = 100000
_INV_EPS = _EPS / (_NC - 1)
_B = 1024
_COEF = (1.0 - _EPS) - _INV_EPS

_SC_ROWS = 768
_TC_ROWS = _B - _SC_ROWS

_TAIL = 32
_TAIL_START = _NC - _TAIL

_BLK_R = 64


def _sum_body(x_ref, strip_ref, tgt_ref, o_ref):
    @pl.when(pl.program_id(0) == 0)
    def _init():
        o_ref[0, 0] = 0.0
        o_ref[0, 1] = 0.0
        col = lax.broadcasted_iota(jnp.int32, (_B, 128), 1) + _TAIL_START
        strip = jnp.where(col < _NC, strip_ref[...], 0.0)
        o_ref[0, 0] += jnp.sum(strip[_TC_ROWS:, :])
        hit = col == tgt_ref[...]
        o_ref[0, 1] += jnp.sum(jnp.where(hit, strip, 0.0))

    o_ref[0, 0] += jnp.sum(x_ref[...])


def _dense_sum(x2d, tgt2d):
    return pl.pallas_call(
        _sum_body,
        grid=(_TC_ROWS // _BLK_R,),
        in_specs=[
            pl.BlockSpec((_BLK_R, _NC), lambda i: (i, 0)),
            pl.BlockSpec((_B, 128), lambda i: (0, (_NC - _TAIL) // 128)),
            pl.BlockSpec((_B, 1), lambda i: (0, 0)),
        ],
        out_specs=pl.BlockSpec((1, 2), lambda i: (0, 0), memory_space=pltpu.SMEM),
        out_shape=jax.ShapeDtypeStruct((1, 2), jnp.float32),
    )(x2d, x2d, tgt2d)


_info = plsc.get_sparse_core_info()
_NCORES = _info.num_cores
_NSUB = _info.num_subcores
_NW = _NCORES * _NSUB
_RPW = _B // _NW
_L = 16

_TW = 128
_TH = 8
_MAXC0 = _NC - _TAIL - _TW

_SC_RPW = _SC_ROWS // _NW
_CHUNK = 1408
_NCH = _TAIL_START // _CHUNK
_NGRP = _SC_RPW // _TH
_T = _NGRP * _NCH


def _sc_body(pred_hbm, tgt_hbm, out_hbm, tgt_v, win_v, part_v, buf0, buf1,
             gsem, sem0, sem1):
    wid = lax.axis_index("s") * _NCORES + lax.axis_index("c")
    base = wid * _RPW
    pltpu.sync_copy(tgt_hbm.at[pl.ds(base, _RPW)], tgt_v)
    iota = lax.iota(jnp.int32, _L)
    copies = []
    scalars = []
    tvecs = [tgt_v[pl.ds(c * _L, _L)] for c in range(_RPW // _L)]
    for r in range(_RPW):
        t = tvecs[r // _L][r % _L]
        c0 = pl.multiple_of(jnp.minimum(jnp.bitwise_and(t, -_TW), _MAXC0), _TW)
        tile_row = pl.multiple_of(base + (r // _TH) * _TH, _TH)
        copies.append(
            pltpu.async_copy(
                pred_hbm.at[pl.ds(tile_row, _TH), pl.ds(c0, _TW)],
                win_v.at[r],
                gsem,
            )
        )
        scalars.append((t, c0))

    row0 = _TC_ROWS + wid * _SC_RPW

    def _src(t):
        r = pl.multiple_of(row0 + (t // _NCH) * _TH, _TH)
        c = pl.multiple_of((t % _NCH) * _CHUNK, _TW)
        return pred_hbm.at[pl.ds(r, _TH), pl.ds(c, _CHUNK)]

    def _acc_chunk(accs, buf, valid):
        def jbody(j, a):
            a0, a1, a2, a3 = a
            col = j * 32
            for h in range(2):
                a0 = a0 + buf[0, pl.ds(col + h * _L, _L)] + buf[4, pl.ds(col + h * _L, _L)]
                a1 = a1 + buf[1, pl.ds(col + h * _L, _L)] + buf[5, pl.ds(col + h * _L, _L)]
                a2 = a2 + buf[2, pl.ds(col + h * _L, _L)] + buf[6, pl.ds(col + h * _L, _L)]
                a3 = a3 + buf[3, pl.ds(col + h * _L, _L)] + buf[7, pl.ds(col + h * _L, _L)]
            return (a0, a1, a2, a3)

        z = jnp.zeros((_L,), jnp.float32)
        d0, d1, d2, d3 = lax.fori_loop(0, _CHUNK // 32, jbody, (z, z, z, z))
        a0, a1, a2, a3 = accs
        return (a0 + valid * d0, a1 + valid * d1, a2 + valid * d2, a3 + valid * d3)

    pltpu.async_copy(_src(0), buf0, sem0)

    def obody(i, accs):
        t0 = 2 * i
        t1 = t0 + 1
        pltpu.async_copy(_src(jnp.minimum(t1, _T - 1)), buf1, sem1)
        pltpu.make_async_copy(_src(0), buf0, sem0).wait()
        accs = _acc_chunk(accs, buf0, jnp.float32(1.0))
        pltpu.async_copy(_src(jnp.minimum(t0 + 2, _T - 1)), buf0, sem0)
        pltpu.make_async_copy(_src(0), buf1, sem1).wait()
        valid = jnp.where(t1 < _T, 1.0, 0.0).astype(jnp.float32)
        return _acc_chunk(accs, buf1, valid)

    z = jnp.zeros((_L,), jnp.float32)
    a0, a1, a2, a3 = lax.fori_loop(0, (_T + 1) // 2, obody, (z, z, z, z))
    pltpu.make_async_copy(_src(0), buf0, sem0).wait()
    dense = (a0 + a1) + (a2 + a3)

    for cp in copies:
        cp.wait()
    gacc = jnp.zeros((_L,), jnp.float32)
    for r in range(_RPW):
        t, c0 = scalars[r]
        lane = jnp.minimum(t - c0, _TW - 1)
        chunk = jnp.bitwise_and(lane, -_L)
        v16 = win_v[r, r % _TH, pl.ds(chunk, _L)]
        hit = jnp.where(iota == lane - chunk, v16, 0.0)
        gacc = gacc + hit * jnp.where(t < _TAIL_START, 1.0, 0.0)
    part_v[...] = jnp.float32(_INV_EPS) * dense + jnp.float32(_COEF) * gacc
    pltpu.sync_copy(part_v, out_hbm.at[wid])


_sc_part = functools.partial(
    pl.kernel,
    mesh=plsc.VectorSubcoreMesh(core_axis_name="c", subcore_axis_name="s"),
    out_type=jax.ShapeDtypeStruct((_NW, _L), jnp.float32),
    scratch_types=[
        pltpu.VMEM((_RPW,), jnp.int32),
        pltpu.VMEM((_RPW, _TH, _TW), jnp.float32),
        pltpu.VMEM((_L,), jnp.float32),
        pltpu.VMEM((_TH, _CHUNK), jnp.float32),
        pltpu.VMEM((_TH, _CHUNK), jnp.float32),
        pltpu.SemaphoreType.DMA,
        pltpu.SemaphoreType.DMA,
        pltpu.SemaphoreType.DMA,
    ],
)(_sc_body)


def kernel(predictions, targets):
    sums = _dense_sum(predictions, targets.reshape(_B, 1))
    parts = _sc_part(predictions, targets)
    return -(_INV_EPS * sums[0, 0] + _COEF * sums[0, 1] + jnp.sum(parts)) / _B


# speedup vs baseline: 1.0009x; 1.0009x over previous
import functools

import jax
import jax.numpy as jnp
from jax import lax
from jax.experimental import pallas as pl
from jax.experimental.pallas import tpu as pltpu
from jax.experimental.pallas import tpu_sc as plsc

_EPS = 0.1
_NC = 100000
_INV_EPS = _EPS / (_NC - 1)
_B = 1024
_COEF = (1.0 - _EPS) - _INV_EPS

_SC_ROWS = 768
_TC_ROWS = _B - _SC_ROWS

_TAIL = 32
_TAIL_START = _NC - _TAIL

_BLK_R = 64


def _sum_body(x_ref, strip_ref, tgt_ref, o_ref):
    @pl.when(pl.program_id(0) == 0)
    def _init():
        o_ref[0, 0] = 0.0
        o_ref[0, 1] = 0.0
        col = lax.broadcasted_iota(jnp.int32, (_B, 128), 1) + _TAIL_START
        strip = jnp.where(col < _NC, strip_ref[...], 0.0)
        o_ref[0, 0] += jnp.sum(strip[_TC_ROWS:, :])
        hit = col == tgt_ref[...]
        o_ref[0, 1] += jnp.sum(jnp.where(hit, strip, 0.0))

    o_ref[0, 0] += jnp.sum(x_ref[...])


def _dense_sum(x2d, tgt2d):
    return pl.pallas_call(
        _sum_body,
        grid=(_TC_ROWS // _BLK_R,),
        in_specs=[
            pl.BlockSpec((_BLK_R, _NC), lambda i: (i, 0)),
            pl.BlockSpec((_B, 128), lambda i: (0, (_NC - _TAIL) // 128)),
            pl.BlockSpec((_B, 1), lambda i: (0, 0)),
        ],
        out_specs=pl.BlockSpec((1, 2), lambda i: (0, 0), memory_space=pltpu.SMEM),
        out_shape=jax.ShapeDtypeStruct((1, 2), jnp.float32),
    )(x2d, x2d, tgt2d)


_info = plsc.get_sparse_core_info()
_NCORES = _info.num_cores
_NSUB = _info.num_subcores
_NW = _NCORES * _NSUB
_RPW = _B // _NW
_L = 16

_TW = 128
_TH = 8
_MAXC0 = _NC - _TAIL - _TW

_SC_RPW = _SC_ROWS // _NW
_CHUNK = 1408
_NCH = _TAIL_START // _CHUNK
_NGRP = _SC_RPW // _TH
_T = _NGRP * _NCH


def _sc_body(pred_hbm, tgt_hbm, out_hbm, tgt_v, win_v, part_v, buf0, buf1,
             gsem, sem0, sem1):
    wid = lax.axis_index("s") * _NCORES + lax.axis_index("c")
    base = wid * _RPW
    pltpu.sync_copy(tgt_hbm.at[pl.ds(base, _RPW)], tgt_v)
    iota = lax.iota(jnp.int32, _L)
    copies = []
    scalars = []
    tvecs = [tgt_v[pl.ds(c * _L, _L)] for c in range(_RPW // _L)]
    for r in range(_RPW):
        t = tvecs[r // _L][r % _L]
        c0 = pl.multiple_of(jnp.minimum(jnp.bitwise_and(t, -_TW), _MAXC0), _TW)
        tile_row = pl.multiple_of(base + (r // _TH) * _TH, _TH)
        copies.append(
            pltpu.async_copy(
                pred_hbm.at[pl.ds(tile_row, _TH), pl.ds(c0, _TW)],
                win_v.at[r],
                gsem,
            )
        )
        scalars.append((t, c0))

    row0 = _TC_ROWS + wid * _SC_RPW

    def _src(t):
        r = pl.multiple_of(row0 + (t // _NCH) * _TH, _TH)
        c = pl.multiple_of((t % _NCH) * _CHUNK, _TW)
        return pred_hbm.at[pl.ds(r, _TH), pl.ds(c, _CHUNK)]

    def _acc_chunk(accs, buf, valid):
        def jbody(j, a):
            a0, a1, a2, a3 = a
            col = j * 32
            for h in range(2):
                a0 = a0 + buf[0, pl.ds(col + h * _L, _L)] + buf[4, pl.ds(col + h * _L, _L)]
                a1 = a1 + buf[1, pl.ds(col + h * _L, _L)] + buf[5, pl.ds(col + h * _L, _L)]
                a2 = a2 + buf[2, pl.ds(col + h * _L, _L)] + buf[6, pl.ds(col + h * _L, _L)]
                a3 = a3 + buf[3, pl.ds(col + h * _L, _L)] + buf[7, pl.ds(col + h * _L, _L)]
            return (a0, a1, a2, a3)

        z = jnp.zeros((_L,), jnp.float32)
        d0, d1, d2, d3 = lax.fori_loop(0, _CHUNK // 32, jbody, (z, z, z, z))
        a0, a1, a2, a3 = accs
        return (a0 + valid * d0, a1 + valid * d1, a2 + valid * d2, a3 + valid * d3)

    pltpu.async_copy(_src(0), buf0, sem0)

    def obody(i, accs):
        t0 = 2 * i
        t1 = t0 + 1
        pltpu.async_copy(_src(jnp.minimum(t1, _T - 1)), buf1, sem1)
        pltpu.make_async_copy(_src(0), buf0, sem0).wait()
        accs = _acc_chunk(accs, buf0, jnp.float32(1.0))
        pltpu.async_copy(_src(jnp.minimum(t0 + 2, _T - 1)), buf0, sem0)
        pltpu.make_async_copy(_src(0), buf1, sem1).wait()
        valid = jnp.where(t1 < _T, 1.0, 0.0).astype(jnp.float32)
        return _acc_chunk(accs, buf1, valid)

    z = jnp.zeros((_L,), jnp.float32)
    a0, a1, a2, a3 = lax.fori_loop(0, (_T + 1) // 2, obody, (z, z, z, z))
    pltpu.make_async_copy(_src(0), buf0, sem0).wait()
    dense = (a0 + a1) + (a2 + a3)

    for cp in copies:
        cp.wait()
    gacc = jnp.zeros((_L,), jnp.float32)
    for r in range(_RPW):
        t, c0 = scalars[r]
        lane = jnp.minimum(t - c0, _TW - 1)
        chunk = jnp.bitwise_and(lane, -_L)
        v16 = win_v[r, r % _TH, pl.ds(chunk, _L)]
        hit = jnp.where(iota == lane - chunk, v16, 0.0)
        gacc = gacc + hit * jnp.where(t < _TAIL_START, 1.0, 0.0)
    part_v[...] = jnp.float32(_INV_EPS) * dense + jnp.float32(_COEF) * gacc
    pltpu.sync_copy(part_v, out_hbm.at[wid])


_sc_part = functools.partial(
    pl.kernel,
    mesh=plsc.VectorSubcoreMesh(core_axis_name="c", subcore_axis_name="s"),
    compiler_params=pltpu.CompilerParams(use_tc_tiling_on_sc=True),
    out_type=jax.ShapeDtypeStruct((_NW, _L), jnp.float32),
    scratch_types=[
        pltpu.VMEM((_RPW,), jnp.int32),
        pltpu.VMEM((_RPW, _TH, _TW), jnp.float32),
        pltpu.VMEM((_L,), jnp.float32),
        pltpu.VMEM((_TH, _CHUNK), jnp.float32),
        pltpu.VMEM((_TH, _CHUNK), jnp.float32),
        pltpu.SemaphoreType.DMA,
        pltpu.SemaphoreType.DMA,
        pltpu.SemaphoreType.DMA,
    ],
)(_sc_body)


def kernel(predictions, targets):
    sums = _dense_sum(predictions, targets.reshape(_B, 1))
    parts = _sc_part(predictions, targets)
    return -(_INV_EPS * sums[0, 0] + _COEF * sums[0, 1] + jnp.sum(parts)) / _B


# speedup vs baseline: 3.6872x; 3.6838x over previous
import functools

import jax
import jax.numpy as jnp
from jax import lax
from jax.experimental import pallas as pl
from jax.experimental.pallas import tpu as pltpu
from jax.experimental.pallas import tpu_sc as plsc

_EPS = 0.1
_NC = 100000
_INV_EPS = _EPS / (_NC - 1)
_B = 1024
_COEF = (1.0 - _EPS) - _INV_EPS

_V0 = 51200

_BLK_V = 6400


def _sum_body(x_ref, o_ref):
    @pl.when(pl.program_id(0) == 0)
    def _init():
        o_ref[0, 0] = 0.0

    o_ref[0, 0] += jnp.sum(x_ref[...])


def _dense_sum(xT):
    return pl.pallas_call(
        _sum_body,
        grid=(_V0 // _BLK_V,),
        in_specs=[pl.BlockSpec((_BLK_V, _B), lambda i: (i, 0))],
        out_specs=pl.BlockSpec((1, 1), lambda i: (0, 0), memory_space=pltpu.SMEM),
        out_shape=jax.ShapeDtypeStruct((1, 1), jnp.float32),
    )(xT)


_info = plsc.get_sparse_core_info()
_NCORES = _info.num_cores
_NSUB = _info.num_subcores
_NW = _NCORES * _NSUB
_RPW = _B // _NW
_L = 16

_TW = 128
_TH = 8

_CH = 32
_NCHUNK = (_NC - _V0) // _CH
_TSUB = -(-_NCHUNK // _NW)


def _sc_body(predT_hbm, tgt_hbm, out_hbm, tgt_v, win_v, part_v, buf0, buf1,
             gsem, sem0, sem1):
    wid = lax.axis_index("s") * _NCORES + lax.axis_index("c")
    base = wid * _RPW
    pltpu.sync_copy(tgt_hbm.at[pl.ds(base, _RPW)], tgt_v)
    iota = lax.iota(jnp.int32, _L)
    copies = []
    scalars = []
    tvecs = [tgt_v[pl.ds(c * _L, _L)] for c in range(_RPW // _L)]
    for r in range(_RPW):
        t = tvecs[r // _L][r % _L]
        v0 = pl.multiple_of(jnp.bitwise_and(t, -_TH), _TH)
        b0 = pl.multiple_of(jnp.bitwise_and(jnp.int32(base + r), -_TW), _TW)
        copies.append(
            pltpu.async_copy(
                predT_hbm.at[pl.ds(v0, _TH), pl.ds(b0, _TW)],
                win_v.at[pl.ds(r * _TH, _TH), :],
                gsem,
            )
        )
        scalars.append(t)

    def _src(j):
        row = _V0 + _CH * (wid + _NW * j)
        row = pl.multiple_of(jnp.minimum(row, _NC - _CH), _CH)
        return predT_hbm.at[pl.ds(row, _CH), :]

    def _acc_chunk(accs, buf, valid):
        def jbody(j, a):
            a0, a1, a2, a3 = a
            col = j * 32
            for rr in range(_CH // 4):
                for h in range(2):
                    d = pl.ds(col + h * _L, _L)
                    a0 = a0 + buf[4 * rr + 0, d]
                    a1 = a1 + buf[4 * rr + 1, d]
                    a2 = a2 + buf[4 * rr + 2, d]
                    a3 = a3 + buf[4 * rr + 3, d]
            return (a0, a1, a2, a3)

        z = jnp.zeros((_L,), jnp.float32)
        d0, d1, d2, d3 = lax.fori_loop(0, _B // 32, jbody, (z, z, z, z))
        a0, a1, a2, a3 = accs
        return (a0 + valid * d0, a1 + valid * d1, a2 + valid * d2, a3 + valid * d3)

    def _valid(j):
        return jnp.where(wid + _NW * j < _NCHUNK, 1.0, 0.0).astype(jnp.float32)

    pltpu.async_copy(_src(jnp.int32(0)), buf0, sem0)

    def obody(i, accs):
        j0 = 2 * i
        j1 = j0 + 1
        pltpu.async_copy(_src(j1), buf1, sem1)
        pltpu.make_async_copy(_src(jnp.int32(0)), buf0, sem0).wait()
        accs = _acc_chunk(accs, buf0, _valid(j0))
        pltpu.async_copy(_src(j0 + 2), buf0, sem0)
        pltpu.make_async_copy(_src(jnp.int32(0)), buf1, sem1).wait()
        return _acc_chunk(accs, buf1, _valid(j1))

    z = jnp.zeros((_L,), jnp.float32)
    a0, a1, a2, a3 = lax.fori_loop(0, (_TSUB + 1) // 2, obody, (z, z, z, z))
    pltpu.make_async_copy(_src(jnp.int32(0)), buf0, sem0).wait()
    dense = (a0 + a1) + (a2 + a3)

    for cp in copies:
        cp.wait()
    gacc = jnp.zeros((_L,), jnp.float32)
    for r in range(_RPW):
        t = scalars[r]
        rowin = jnp.bitwise_and(t, _TH - 1)
        lane = jnp.bitwise_and(base + r, _TW - 1)
        chunk = jnp.bitwise_and(lane, -_L)
        v16 = win_v[r * _TH + rowin, pl.ds(chunk, _L)]
        gacc = gacc + jnp.where(iota == lane - chunk, v16, 0.0)
    part_v[...] = jnp.float32(_INV_EPS) * dense + jnp.float32(_COEF) * gacc
    pltpu.sync_copy(part_v, out_hbm.at[wid])


_sc_part = functools.partial(
    pl.kernel,
    mesh=plsc.VectorSubcoreMesh(core_axis_name="c", subcore_axis_name="s"),
    out_type=jax.ShapeDtypeStruct((_NW, _L), jnp.float32),
    scratch_types=[
        pltpu.VMEM((_RPW,), jnp.int32),
        pltpu.VMEM((_RPW * _TH, _TW), jnp.float32),
        pltpu.VMEM((_L,), jnp.float32),
        pltpu.VMEM((_CH, _B), jnp.float32),
        pltpu.VMEM((_CH, _B), jnp.float32),
        pltpu.SemaphoreType.DMA,
        pltpu.SemaphoreType.DMA,
        pltpu.SemaphoreType.DMA,
    ],
)(_sc_body)


def kernel(predictions, targets):
    predT = predictions.T
    total = _dense_sum(predT)[0, 0]
    parts = _sc_part(predT, targets)
    return -(_INV_EPS * total + jnp.sum(parts)) / _B
